# baseline (device time: 112751 ns/iter reference)
import jax
import jax.numpy as jnp
from jax import lax
from jax.experimental import pallas as pl
from jax.experimental.pallas import tpu as pltpu

N_DEV = 4
B, S, D = 1, 1024, 2048
DC = 128
H, DH, DR = 16, 128, 32
SQ = S // N_DEV
SB = 256
SCALE = (DH + DR) ** -0.5
F32 = jnp.float32
BF = jnp.bfloat16

_VMEM = pl.BlockSpec(memory_space=pltpu.VMEM)

_PEER_WAIT_ORDER = (1, 3, 2)


def _signal_peers(sem, my):
    for d in range(1, N_DEV):
        pl.semaphore_signal(
            sem, inc=1,
            device_id=((my + d) % N_DEV,),
            device_id_type=pl.DeviceIdType.MESH,
        )


def _gather_qkv(x, Wdkv, Wuk, Wuv, Wkr, Wq_bf, Wqr_bf):

    def body(x_ref, wdkv_ref, wuk_ref, wuv_ref, wkr_ref, wq_ref, wqr_ref,
             k_ref, v_ref, kr_ref, q_out, qr_out,
             c_full, wuk_full, wuv_full, send_sems, recv_sems):
        my = lax.axis_index("i")
        barrier_sem = pltpu.get_barrier_semaphore()
        _signal_peers(barrier_sem, my)

        wdkv_b = wdkv_ref[...].astype(BF)
        wkr_b = wkr_ref[...].astype(BF)
        for b in range(S // SB):
            xb = x_ref[0, b * SB:(b + 1) * SB, :].astype(BF)
            c_full[b * SB:(b + 1) * SB, pl.ds(my * DC, DC)] = jnp.dot(
                xb, wdkv_b, preferred_element_type=F32).astype(BF)
            kr_ref[b * SB:(b + 1) * SB, :] = jnp.dot(
                xb, wkr_b, preferred_element_type=F32).astype(BF)
        wuk_full[pl.ds(my * DC, DC), :] = wuk_ref[...].astype(BF)
        wuv_full[pl.ds(my * DC, DC), :] = wuv_ref[...].astype(BF)

        pl.semaphore_wait(barrier_sem, N_DEV - 1)

        rdmas = {}
        for d in range(1, N_DEV):
            tgt = (my + d) % N_DEV
            slices = [
                c_full.at[:, pl.ds(my * DC, DC)],
                wuk_full.at[pl.ds(my * DC, DC), :],
                wuv_full.at[pl.ds(my * DC, DC), :],
            ]
            for a, sl in enumerate(slices):
                rdma = pltpu.make_async_remote_copy(
                    src_ref=sl,
                    dst_ref=sl,
                    send_sem=send_sems.at[d - 1, a],
                    recv_sem=recv_sems.at[d - 1, a],
                    device_id=(tgt,),
                    device_id_type=pl.DeviceIdType.MESH,
                )
                rdma.start()
                rdmas[(d, a)] = rdma

        xq = x_ref[0, pl.ds(my * SQ, SQ), :].astype(BF)
        q_out[...] = jnp.dot(xq, wq_ref[...],
                             preferred_element_type=F32).astype(BF)
        qr_out[...] = jnp.dot(xq, wqr_ref[...],
                              preferred_element_type=F32).astype(BF)

        def partial(j):
            cj = c_full[:, pl.ds(j * DC, DC)]
            wk = wuk_full[pl.ds(j * DC, DC), :]
            wv = wuv_full[pl.ds(j * DC, DC), :]
            return (jnp.dot(cj, wk, preferred_element_type=F32).astype(BF),
                    jnp.dot(cj, wv, preferred_element_type=F32).astype(BF))

        kp, vp = partial(my)
        k_ref[...] = kp
        v_ref[...] = vp
        for d in _PEER_WAIT_ORDER:
            for a in range(3):
                rdmas[(d, a)].wait_recv()
            kp, vp = partial((my + N_DEV - d) % N_DEV)
            k_ref[...] = k_ref[...] + kp
            v_ref[...] = v_ref[...] + vp
        for d in range(1, N_DEV):
            for a in range(3):
                rdmas[(d, a)].wait_send()

    return pl.pallas_call(
        body,
        out_shape=[
            jax.ShapeDtypeStruct((S, H * DH), BF),
            jax.ShapeDtypeStruct((S, H * DH), BF),
            jax.ShapeDtypeStruct((S, DR), BF),
            jax.ShapeDtypeStruct((SQ, H * DH), BF),
            jax.ShapeDtypeStruct((SQ, H * DR), BF),
        ],
        in_specs=[_VMEM] * 7,
        out_specs=[_VMEM] * 5,
        scratch_shapes=[
            pltpu.VMEM((S, N_DEV * DC), BF),
            pltpu.VMEM((N_DEV * DC, D), BF),
            pltpu.VMEM((N_DEV * DC, D), BF),
            pltpu.SemaphoreType.DMA((N_DEV - 1, 3)),
            pltpu.SemaphoreType.DMA((N_DEV - 1, 3)),
        ],
        compiler_params=pltpu.CompilerParams(collective_id=0),
    )(x, Wdkv, Wuk, Wuv, Wkr, Wq_bf, Wqr_bf)


def _attn_out_bcast(q, qr, kr, K, V, Wo_bf):

    def body(q_ref, qr_ref, kr_ref, k_ref, v_ref, wo_ref,
             out_ref, out_bf, send_sems, recv_sems):
        my = lax.axis_index("i")
        q0 = my * SQ
        barrier_sem = pltpu.get_barrier_semaphore()
        _signal_peers(barrier_sem, my)

        kr = kr_ref[...]
        outs = []
        for h in range(H):
            qh = q_ref[:, h * DH:(h + 1) * DH]
            kh = k_ref[:, h * DH:(h + 1) * DH]
            vh = v_ref[:, h * DH:(h + 1) * DH]
            qrh = qr_ref[:, h * DR:(h + 1) * DR]
            s = lax.dot_general(qh, kh, (((1,), (1,)), ((), ())),
                                preferred_element_type=F32)
            s = s + lax.dot_general(qrh, kr, (((1,), (1,)), ((), ())),
                                    preferred_element_type=F32)
            s = s * SCALE
            m = jnp.max(s, axis=1, keepdims=True)
            p = jnp.exp(s - m)
            p = (p / jnp.sum(p, axis=1, keepdims=True)).astype(BF)
            outs.append(jnp.dot(p, vh, preferred_element_type=F32).astype(BF))
        o = jnp.concatenate(outs, axis=1)

        out_q = jnp.dot(o, wo_ref[...], preferred_element_type=F32)
        out_ref[0, pl.ds(q0, SQ), :] = out_q
        out_bf[pl.ds(q0, SQ), :] = out_q.astype(BF)

        pl.semaphore_wait(barrier_sem, N_DEV - 1)
        sends = []
        for d in range(1, N_DEV):
            tgt = (my + d) % N_DEV
            sl = out_bf.at[pl.ds(q0, SQ), :]
            rdma = pltpu.make_async_remote_copy(
                src_ref=sl,
                dst_ref=sl,
                send_sem=send_sems.at[d - 1],
                recv_sem=recv_sems.at[d - 1],
                device_id=(tgt,),
                device_id_type=pl.DeviceIdType.MESH,
            )
            rdma.start()
            sends.append(rdma)

        for rdma in sends:
            rdma.wait_recv()
        for d in range(1, N_DEV):
            p0 = ((my + N_DEV - d) % N_DEV) * SQ
            out_ref[0, pl.ds(p0, SQ), :] = out_bf[pl.ds(p0, SQ), :].astype(F32)
        for rdma in sends:
            rdma.wait_send()

    return pl.pallas_call(
        body,
        out_shape=jax.ShapeDtypeStruct((B, S, D), F32),
        in_specs=[_VMEM] * 6,
        out_specs=_VMEM,
        scratch_shapes=[
            pltpu.VMEM((S, D), BF),
            pltpu.SemaphoreType.DMA((N_DEV - 1,)),
            pltpu.SemaphoreType.DMA((N_DEV - 1,)),
        ],
        compiler_params=pltpu.CompilerParams(collective_id=1),
    )(q, qr, kr, K, V, Wo_bf)


def kernel(x, Wdkv, Wuk, Wuv, Wq, Wqr, Wkr, Wo):
    K, V, kr, q, qr = _gather_qkv(
        x, Wdkv, Wuk, Wuv, Wkr, Wq.astype(BF), Wqr.astype(BF))
    return _attn_out_bcast(q, qr, kr, K, V, Wo.astype(BF))


# device time: 98149 ns/iter; 1.1488x vs baseline; 1.1488x over previous
import jax
import jax.numpy as jnp
from jax import lax
from jax.experimental import pallas as pl
from jax.experimental.pallas import tpu as pltpu

N_DEV = 4
B, S, D = 1, 1024, 2048
DC = 128
H, DH, DR = 16, 128, 32
SQ = S // N_DEV
SB = 256
SH = SQ // 2
SCALE = (DH + DR) ** -0.5
F32 = jnp.float32
BF = jnp.bfloat16

_VMEM = pl.BlockSpec(memory_space=pltpu.VMEM)

_PEER_WAIT_ORDER = (1, 3, 2)


def _signal_peers(sem, my):
    for d in range(1, N_DEV):
        pl.semaphore_signal(
            sem, inc=1,
            device_id=((my + d) % N_DEV,),
            device_id_type=pl.DeviceIdType.MESH,
        )


def _gather_qkv(x, Wdkv, Wuk, Wuv, Wkr, Wq, Wqr):

    def body(x_ref, wdkv_ref, wuk_ref, wuv_ref, wkr_ref, wq_ref, wqr_ref,
             k_ref, v_ref, kr_ref, q_out, qr_out,
             c_full, wuk_full, wuv_full, send_sems, recv_sems):
        my = lax.axis_index("i")
        barrier_sem = pltpu.get_barrier_semaphore()
        _signal_peers(barrier_sem, my)

        def remote_copy(sl, d, a, tgt):
            return pltpu.make_async_remote_copy(
                src_ref=sl,
                dst_ref=sl,
                send_sem=send_sems.at[d - 1, a],
                recv_sem=recv_sems.at[d - 1, a],
                device_id=(tgt,),
                device_id_type=pl.DeviceIdType.MESH,
            )

        wuk_full[pl.ds(my * DC, DC), :] = wuk_ref[...].astype(BF)
        wuv_full[pl.ds(my * DC, DC), :] = wuv_ref[...].astype(BF)
        pl.semaphore_wait(barrier_sem, N_DEV - 1)
        rdmas = {}
        for d in range(1, N_DEV):
            tgt = (my + d) % N_DEV
            rdmas[(d, 1)] = remote_copy(
                wuk_full.at[pl.ds(my * DC, DC), :], d, 1, tgt)
            rdmas[(d, 2)] = remote_copy(
                wuv_full.at[pl.ds(my * DC, DC), :], d, 2, tgt)
            rdmas[(d, 1)].start()
            rdmas[(d, 2)].start()

        wdkv_b = wdkv_ref[...].astype(BF)
        wkr_b = wkr_ref[...].astype(BF)
        for b in range(S // SB):
            xb = x_ref[0, b * SB:(b + 1) * SB, :].astype(BF)
            c_full[b * SB:(b + 1) * SB, pl.ds(my * DC, DC)] = jnp.dot(
                xb, wdkv_b, preferred_element_type=F32).astype(BF)
            kr_ref[b * SB:(b + 1) * SB, :] = jnp.dot(
                xb, wkr_b, preferred_element_type=F32).astype(BF)
        for d in range(1, N_DEV):
            tgt = (my + d) % N_DEV
            rdmas[(d, 0)] = remote_copy(
                c_full.at[:, pl.ds(my * DC, DC)], d, 0, tgt)
            rdmas[(d, 0)].start()

        xq = x_ref[0, pl.ds(my * SQ, SQ), :]
        q_out[...] = jnp.dot(xq, wq_ref[...],
                             preferred_element_type=F32).astype(BF)
        qr_out[...] = jnp.dot(xq, wqr_ref[...],
                              preferred_element_type=F32).astype(BF)

        def partial(j):
            cj = c_full[:, pl.ds(j * DC, DC)]
            wk = wuk_full[pl.ds(j * DC, DC), :]
            wv = wuv_full[pl.ds(j * DC, DC), :]
            return (jnp.dot(cj, wk, preferred_element_type=F32).astype(BF),
                    jnp.dot(cj, wv, preferred_element_type=F32).astype(BF))

        kp, vp = partial(my)
        k_ref[...] = kp
        v_ref[...] = vp
        for d in _PEER_WAIT_ORDER:
            for a in range(3):
                rdmas[(d, a)].wait_recv()
            kp, vp = partial((my + N_DEV - d) % N_DEV)
            k_ref[...] = k_ref[...] + kp
            v_ref[...] = v_ref[...] + vp
        for d in range(1, N_DEV):
            for a in range(3):
                rdmas[(d, a)].wait_send()

    return pl.pallas_call(
        body,
        out_shape=[
            jax.ShapeDtypeStruct((S, H * DH), BF),
            jax.ShapeDtypeStruct((S, H * DH), BF),
            jax.ShapeDtypeStruct((S, DR), BF),
            jax.ShapeDtypeStruct((SQ, H * DH), BF),
            jax.ShapeDtypeStruct((SQ, H * DR), BF),
        ],
        in_specs=[_VMEM] * 7,
        out_specs=[_VMEM] * 5,
        scratch_shapes=[
            pltpu.VMEM((S, N_DEV * DC), BF),
            pltpu.VMEM((N_DEV * DC, D), BF),
            pltpu.VMEM((N_DEV * DC, D), BF),
            pltpu.SemaphoreType.DMA((N_DEV - 1, 3)),
            pltpu.SemaphoreType.DMA((N_DEV - 1, 3)),
        ],
        compiler_params=pltpu.CompilerParams(collective_id=0),
    )(x, Wdkv, Wuk, Wuv, Wkr, Wq, Wqr)


def _attn_out_bcast(q, qr, kr, K, V, Wo):

    def body(q_ref, qr_ref, kr_ref, k_ref, v_ref, wo_ref,
             out_ref, out_bf, send_sems, recv_sems):
        my = lax.axis_index("i")
        q0 = my * SQ
        barrier_sem = pltpu.get_barrier_semaphore()
        _signal_peers(barrier_sem, my)

        kr = kr_ref[...]
        barrier_done = False
        rdmas = {}
        for c in range(2):
            r0 = c * SH
            outs = []
            for h in range(H):
                qh = q_ref[r0:r0 + SH, h * DH:(h + 1) * DH]
                kh = k_ref[:, h * DH:(h + 1) * DH]
                vh = v_ref[:, h * DH:(h + 1) * DH]
                qrh = qr_ref[r0:r0 + SH, h * DR:(h + 1) * DR]
                s = lax.dot_general(qh, kh, (((1,), (1,)), ((), ())),
                                    preferred_element_type=F32)
                s = s + lax.dot_general(qrh, kr, (((1,), (1,)), ((), ())),
                                        preferred_element_type=F32)
                s = s * SCALE
                m = jnp.max(s, axis=1, keepdims=True)
                p = jnp.exp(s - m)
                p = (p / jnp.sum(p, axis=1, keepdims=True)).astype(BF)
                outs.append(jnp.dot(p, vh, preferred_element_type=F32)
                            .astype(BF))
            o = jnp.concatenate(outs, axis=1)

            out_h = jnp.dot(o.astype(F32), wo_ref[...],
                            preferred_element_type=F32)
            out_ref[0, pl.ds(q0 + r0, SH), :] = out_h
            out_bf[pl.ds(q0 + r0, SH), :] = out_h.astype(BF)

            if not barrier_done:
                pl.semaphore_wait(barrier_sem, N_DEV - 1)
                barrier_done = True
            for d in range(1, N_DEV):
                tgt = (my + d) % N_DEV
                sl = out_bf.at[pl.ds(q0 + r0, SH), :]
                rdma = pltpu.make_async_remote_copy(
                    src_ref=sl,
                    dst_ref=sl,
                    send_sem=send_sems.at[d - 1, c],
                    recv_sem=recv_sems.at[d - 1, c],
                    device_id=(tgt,),
                    device_id_type=pl.DeviceIdType.MESH,
                )
                rdma.start()
                rdmas[(d, c)] = rdma

        for d in range(1, N_DEV):
            for c in range(2):
                rdmas[(d, c)].wait_recv()
        for d in range(1, N_DEV):
            p0 = ((my + N_DEV - d) % N_DEV) * SQ
            out_ref[0, pl.ds(p0, SQ), :] = out_bf[pl.ds(p0, SQ), :].astype(F32)
        for d in range(1, N_DEV):
            for c in range(2):
                rdmas[(d, c)].wait_send()

    return pl.pallas_call(
        body,
        out_shape=jax.ShapeDtypeStruct((B, S, D), F32),
        in_specs=[_VMEM] * 6,
        out_specs=_VMEM,
        scratch_shapes=[
            pltpu.VMEM((S, D), BF),
            pltpu.SemaphoreType.DMA((N_DEV - 1, 2)),
            pltpu.SemaphoreType.DMA((N_DEV - 1, 2)),
        ],
        compiler_params=pltpu.CompilerParams(collective_id=1),
    )(q, qr, kr, K, V, Wo)


def kernel(x, Wdkv, Wuk, Wuv, Wq, Wqr, Wkr, Wo):
    K, V, kr, q, qr = _gather_qkv(x, Wdkv, Wuk, Wuv, Wkr, Wq, Wqr)
    return _attn_out_bcast(q, qr, kr, K, V, Wo)
